# baseline (speedup 1.0000x reference)
import functools

import jax
import jax.numpy as jnp
from jax import lax
from jax.experimental import pallas as pl
from jax.experimental.pallas import tpu as pltpu
from jax.experimental.pallas import tpu_sc as plsc

_B, _S, _D = 16, 4096, 1024
_NSUB = 16
_DH = _D // 2
_T = 32
_LANES = 16
_VPD = _DH // _LANES


def _pool_body(hs, lens, out, len_v, inv_v, buf, acc, rbuf, obuf, stage_sh):
    c = lax.axis_index("c")
    s = lax.axis_index("s")

    pltpu.sync_copy(lens, len_v.at[pl.ds(0, _B)])
    inv_v[pl.ds(0, _B)] = 1.0 / len_v[pl.ds(0, _B)].astype(jnp.float32)

    def zbody(k, carry):
        acc[0, pl.ds(k * _LANES, _LANES)] = jnp.zeros((_LANES,), jnp.float32)
        return carry

    lax.fori_loop(0, _B * _VPD, zbody, 0)

    def batch_body(i, bcarry):
        L = len_v[pl.ds(i, _LANES)][0]
        start = 1 + (s * L) // _NSUB
        end = 1 + ((s + 1) * L) // _NSUB
        cnt = end - start
        nf = cnt // _T
        rem = cnt - nf * _T

        def accrows(lo, hi):
            def rbody(r, carry):
                for v in range(_VPD):
                    plsc.addupdate(
                        acc.at[0, pl.ds(i * _DH + v * _LANES, _LANES)],
                        buf[r, pl.ds(v * _LANES, _LANES)])
                return carry
            lax.fori_loop(lo, hi, rbody, 0)

        def cbody(k, carry):
            pltpu.sync_copy(hs.at[i, pl.ds(start + k * _T, _T), c], buf)
            accrows(0, _T)
            return carry

        lax.fori_loop(0, nf, cbody, 0)

        @pl.when(rem > 0)
        def _():
            tst_raw = start + nf * _T
            tst = jnp.minimum(tst_raw, _S - _T)
            lo = tst_raw - tst
            hi = start + cnt - tst
            pltpu.sync_copy(hs.at[i, pl.ds(tst, _T), c], buf)
            accrows(lo, hi)

        return bcarry

    lax.fori_loop(0, _B, batch_body, 0)

    pltpu.sync_copy(acc, stage_sh.at[pl.ds(s, 1)])
    plsc.subcore_barrier()

    pltpu.sync_copy(
        stage_sh.at[pl.ds(0, _NSUB), pl.ds(s * _DH, _DH)], rbuf)
    inv = inv_v[pl.ds(s, _LANES)][0]

    def redbody(v, carry):
        sl = pl.ds(v * _LANES, _LANES)
        x = rbuf[0, sl]
        for t in range(1, _NSUB):
            x = x + rbuf[t, sl]
        obuf[0, sl] = x * inv
        return carry

    lax.fori_loop(0, _VPD, redbody, 0)
    pltpu.sync_copy(obuf, out.at[pl.ds(s, 1), c])


@jax.jit
def kernel(hidden_states, pivot_len_list):
    hs = hidden_states.reshape(_B, _S, 2, _DH)
    mesh = plsc.VectorSubcoreMesh(core_axis_name="c", subcore_axis_name="s")
    pool = functools.partial(
        pl.kernel,
        out_type=jax.ShapeDtypeStruct((_B, 2, _DH), jnp.float32),
        mesh=mesh,
        scratch_types=[
            pltpu.VMEM((2 * _LANES,), jnp.int32),
            pltpu.VMEM((2 * _LANES,), jnp.float32),
            pltpu.VMEM((_T, _DH), jnp.float32),
            pltpu.VMEM((1, _B * _DH), jnp.float32),
            pltpu.VMEM((_NSUB, _DH), jnp.float32),
            pltpu.VMEM((1, _DH), jnp.float32),
            pltpu.VMEM_SHARED((_NSUB, _B * _DH), jnp.float32),
        ],
    )(_pool_body)
    out = pool(hs, pivot_len_list)
    return out.reshape(_B, _D)


# speedup vs baseline: 1.3835x; 1.3835x over previous
import functools

import jax
import jax.numpy as jnp
from jax import lax
from jax.experimental import pallas as pl
from jax.experimental.pallas import tpu as pltpu
from jax.experimental.pallas import tpu_sc as plsc

_B, _S, _D = 16, 4096, 1024
_NSUB = 16
_DH = _D // 2
_LANES = 16
_T = 32
_VPD = _DH // _LANES


def _pool_body(hs, lens, out, len_v, inv_v, bufs, tbuf, acc, rbuf, obuf,
               stage_sh, semc, semt):
    c = lax.axis_index("c")
    s = lax.axis_index("s")

    pltpu.sync_copy(lens, len_v.at[pl.ds(0, _B)])
    inv_v[pl.ds(0, _B)] = 1.0 / len_v[pl.ds(0, _B)].astype(jnp.float32)

    zacc = (jnp.zeros((_LANES,), jnp.float32),) * _VPD

    for i in range(_B):
        L = len_v[pl.ds(i, _LANES)][0]
        start = 1 + (s * L) // _NSUB
        cnt = 1 + ((s + 1) * L) // _NSUB - start
        nf = cnt // _T
        rem = cnt - nf * _T

        @pl.when(nf > 0)
        def _():
            pltpu.async_copy(
                hs.at[i, pl.ds(start, _T), c], bufs.at[0], semc)

        tst_raw = start + nf * _T
        tst = jnp.minimum(tst_raw, _S - _T)
        tlo = tst_raw - tst

        @pl.when(rem > 0)
        def _():
            pltpu.async_copy(hs.at[i, pl.ds(tst, _T), c], tbuf, semt)

        def cbody(k, accs):
            p = lax.rem(k, 2)
            pltpu.make_async_copy(
                hs.at[i, pl.ds(start, _T), c], bufs.at[0], semc).wait()

            @pl.when(k + 1 < nf)
            def _():
                pltpu.async_copy(
                    hs.at[i, pl.ds(start + (k + 1) * _T, _T), c],
                    bufs.at[1 - p], semc)

            def rbody(r, a):
                return tuple(
                    a[v] + bufs[p, r, pl.ds(v * _LANES, _LANES)]
                    for v in range(_VPD))

            return lax.fori_loop(0, _T, rbody, accs)

        accs = lax.fori_loop(0, nf, cbody, zacc)

        @pl.when(rem > 0)
        def _():
            pltpu.make_async_copy(
                hs.at[i, pl.ds(tst, _T), c], tbuf, semt).wait()

        def tbody(r, a):
            return tuple(
                a[v] + tbuf[r, pl.ds(v * _LANES, _LANES)]
                for v in range(_VPD))

        accs = lax.fori_loop(tlo, tlo + rem, tbody, accs)

        for v in range(_VPD):
            acc[0, pl.ds(i * _DH + v * _LANES, _LANES)] = accs[v]

    pltpu.sync_copy(acc, stage_sh.at[pl.ds(s, 1)])
    plsc.subcore_barrier()

    pltpu.sync_copy(
        stage_sh.at[pl.ds(0, _NSUB), pl.ds(s * _DH, _DH)], rbuf)
    inv = inv_v[pl.ds(s, _LANES)][0]

    def redbody(v, carry):
        sl = pl.ds(v * _LANES, _LANES)
        x = rbuf[0, sl]
        for t in range(1, _NSUB):
            x = x + rbuf[t, sl]
        obuf[0, sl] = x * inv
        return carry

    lax.fori_loop(0, _VPD, redbody, 0)
    pltpu.sync_copy(obuf, out.at[pl.ds(s, 1), c])


@jax.jit
def kernel(hidden_states, pivot_len_list):
    hs = hidden_states.reshape(_B, _S, 2, _DH)
    mesh = plsc.VectorSubcoreMesh(core_axis_name="c", subcore_axis_name="s")
    pool = functools.partial(
        pl.kernel,
        out_type=jax.ShapeDtypeStruct((_B, 2, _DH), jnp.float32),
        mesh=mesh,
        scratch_types=[
            pltpu.VMEM((2 * _LANES,), jnp.int32),
            pltpu.VMEM((2 * _LANES,), jnp.float32),
            pltpu.VMEM((2, _T, _DH), jnp.float32),
            pltpu.VMEM((_T, _DH), jnp.float32),
            pltpu.VMEM((1, _B * _DH), jnp.float32),
            pltpu.VMEM((_NSUB, _DH), jnp.float32),
            pltpu.VMEM((1, _DH), jnp.float32),
            pltpu.VMEM_SHARED((_NSUB, _B * _DH), jnp.float32),
            pltpu.SemaphoreType.DMA,
            pltpu.SemaphoreType.DMA,
        ],
    )(_pool_body)
    out = pool(hs, pivot_len_list)
    return out.reshape(_B, _D)
